# Initial kernel scaffold; baseline (speedup 1.0000x reference)
import jax
import jax.numpy as jnp
from jax.experimental import pallas as pl


def kernel(mentions, W, b):
    raise NotImplementedError("write your pallas kernel here")


# speedup vs baseline: 5.0407x; 5.0407x over previous
import functools

import jax
import jax.numpy as jnp
from jax.experimental import pallas as pl
from jax.experimental.pallas import tpu as pltpu

_KEY_XOR_MASK = 0x7FFFFFFF
_SENTINEL = -2147483648
_KEY_NEG_INF = -2139095041


def _f32_to_key(x):
    s = jax.lax.bitcast_convert_type(x, jnp.int32)
    flip = jnp.right_shift(s, 31) & _KEY_XOR_MASK
    return s ^ flip


def _key_to_f32(k):
    flip = jnp.right_shift(k, 31) & _KEY_XOR_MASK
    return jax.lax.bitcast_convert_type(k ^ flip, jnp.float32)


def _topk_body(K, R, N, m_rows_ref, m_all_ref, w_ref, b_ref,
               out_v_ref, out_i_ref, keys_ref, topv_ref, topi_ref):
    blk = pl.program_id(0)
    prec = jax.lax.Precision.DEFAULT
    bilin = jax.lax.dot_general(
        m_rows_ref[...], w_ref[...], (((1,), (1,)), ((), ())),
        preferred_element_type=jnp.float32, precision=prec) + b_ref[...]
    scores = jax.lax.dot_general(
        bilin, m_all_ref[...], (((1,), (1,)), ((), ())),
        preferred_element_type=jnp.float32, precision=prec)
    col = jax.lax.broadcasted_iota(jnp.int32, (R, N), 1)
    row = jax.lax.broadcasted_iota(jnp.int32, (R, N), 0) + blk * R
    keys_ref[...] = jnp.where(col < row, _f32_to_key(scores), _KEY_NEG_INF)
    topv_ref[...] = jnp.zeros_like(topv_ref)
    topi_ref[...] = jnp.zeros_like(topi_ref)

    slot = jax.lax.broadcasted_iota(jnp.int32, topv_ref.shape, 1)

    def body(k, _):
        keys = keys_ref[...]
        rowmax = jnp.max(keys, axis=1, keepdims=True)
        eq = keys == rowmax
        idx = jnp.min(jnp.where(eq, col, jnp.int32(N)), axis=1,
                      keepdims=True)
        keys_ref[...] = jnp.where(col == idx, _SENTINEL, keys)
        sel = slot == k
        topv_ref[...] = jnp.where(sel, rowmax, topv_ref[...])
        topi_ref[...] = jnp.where(sel, idx, topi_ref[...])
        return 0

    jax.lax.fori_loop(0, K, body, 0)
    out_v_ref[...] = _key_to_f32(topv_ref[...][:, :K])
    out_i_ref[...] = topi_ref[...][:, :K]


def kernel(mentions, W, b):
    n, f = mentions.shape
    K = min(50, n)
    R = min(256, n)
    assert n % R == 0
    kp = 64 if K <= 64 else ((K + 127) // 128) * 128
    grid = (n // R,)
    body = functools.partial(_topk_body, K, R, n)
    out_v, out_i = pl.pallas_call(
        body,
        grid=grid,
        in_specs=[
            pl.BlockSpec((R, f), lambda i: (i, 0)),
            pl.BlockSpec((n, f), lambda i: (0, 0)),
            pl.BlockSpec((f, f), lambda i: (0, 0)),
            pl.BlockSpec((1, f), lambda i: (0, 0)),
        ],
        out_specs=[
            pl.BlockSpec((R, K), lambda i: (i, 0)),
            pl.BlockSpec((R, K), lambda i: (i, 0)),
        ],
        out_shape=[
            jax.ShapeDtypeStruct((n, K), jnp.float32),
            jax.ShapeDtypeStruct((n, K), jnp.int32),
        ],
        scratch_shapes=[
            pltpu.VMEM((R, n), jnp.int32),
            pltpu.VMEM((R, kp), jnp.int32),
            pltpu.VMEM((R, kp), jnp.int32),
        ],
    )(mentions, mentions, W, b.reshape(1, f))
    return out_v, out_i


# speedup vs baseline: 6.7854x; 1.3461x over previous
import functools

import jax
import jax.numpy as jnp
from jax import lax
from jax.experimental import pallas as pl
from jax.experimental.pallas import tpu as pltpu
from jax.experimental.pallas import tpu_sc as plsc

_KEY_XOR_MASK = 0x7FFFFFFF
_SENTINEL = -2147483648
_KEY_NEG_INF = -2139095041

_NC = 2
_NS = 16
_NW = _NC * _NS
_L = 16

_CAP = 512


def _f32_to_key(x):
    s = lax.bitcast_convert_type(x, jnp.int32)
    flip = jnp.right_shift(s, 31) & _KEY_XOR_MASK
    return s ^ flip


def _key_to_f32(k):
    flip = jnp.right_shift(k, 31) & _KEY_XOR_MASK
    return lax.bitcast_convert_type(k ^ flip, jnp.float32)


def _score_body(K, R, N, G, m_rows_ref, m_all_ref, w_ref, b_ref,
                keys_ref, thr_ref):
    blk = pl.program_id(0)
    prec = lax.Precision.DEFAULT
    bilin = lax.dot_general(
        m_rows_ref[...], w_ref[...], (((1,), (1,)), ((), ())),
        preferred_element_type=jnp.float32, precision=prec) + b_ref[...]
    scores = lax.dot_general(
        bilin, m_all_ref[...], (((1,), (1,)), ((), ())),
        preferred_element_type=jnp.float32, precision=prec)
    col = lax.broadcasted_iota(jnp.int32, (R, N), 1)
    row = lax.broadcasted_iota(jnp.int32, (R, N), 0) + blk * R
    keys = jnp.where(col < row, _f32_to_key(scores), _KEY_NEG_INF)
    keys_ref[...] = keys
    gm = keys[:, 0:G]
    for s in range(1, N // G):
        gm = jnp.maximum(gm, keys[:, s * G:(s + 1) * G])
    giota = lax.broadcasted_iota(jnp.int32, (R, G), 1)

    def tb(_, carry):
        gm, _ = carry
        m = jnp.max(gm, axis=1, keepdims=True)
        idx = jnp.min(jnp.where(gm == m, giota, jnp.int32(G)), axis=1,
                      keepdims=True)
        return jnp.where(giota == idx, _SENTINEL, gm), m

    _, t = lax.fori_loop(0, K, tb, (gm, jnp.full((R, 1), _SENTINEL,
                                                 jnp.int32)))
    thr_ref[...] = jnp.broadcast_to(t, (R, _L))


def _sc_body(K, N, KP, keys_hbm, thr_hbm, outv_hbm, outi_hbm,
             thr_v, row_v, ck, ci, ok, oi, of):
    wid = lax.axis_index("s") * _NC + lax.axis_index("c")
    lanes = lax.iota(jnp.int32, _L)
    lane0 = lanes == 0
    ones = jnp.ones((_L,), jnp.int32)

    def row_body(t, _):
        r = wid + t * _NW
        pltpu.sync_copy(keys_hbm.at[r], row_v)
        pltpu.sync_copy(thr_hbm.at[r], thr_v)
        tvec = thr_v[...]
        rvec = jnp.full((_L,), r, jnp.int32)
        nv = (r + _L - 1) // _L

        def filt(j, ptr):
            v = row_v[pl.ds(j * _L, _L)]
            colv = lanes + j * _L
            m = (v >= tvec) & (colv < rvec)
            s = plsc.cumsum(m.astype(jnp.int32))
            pos = jnp.minimum(jnp.full((_L,), ptr, jnp.int32) + s - 1,
                              _CAP - 1)
            plsc.store_scatter(ck, [pos], v, mask=m)
            plsc.store_scatter(ci, [pos], colv, mask=m)
            return ptr + jnp.max(s)

        cnt = lax.fori_loop(0, nv, filt, jnp.int32(0))
        plsc.store_scatter(
            ck, [jnp.minimum(jnp.full((_L,), cnt, jnp.int32) + lanes,
                             _CAP - 1)],
            jnp.full((_L,), _SENTINEL, jnp.int32), mask=ones > 0)
        nvv = jnp.minimum(cnt // _L + 1, _CAP // _L)

        def sel(k, _):
            def mx(j, acc):
                return jnp.maximum(acc, ck[pl.ds(j * _L, _L)])

            mvec = lax.fori_loop(0, nvv, mx,
                                 jnp.full((_L,), _SENTINEL, jnp.int32))
            g = jnp.max(mvec)
            gv = jnp.full((_L,), g, jnp.int32)

            def fp(j, acc):
                v = ck[pl.ds(j * _L, _L)]
                p = jnp.where(v == gv, lanes + j * _L, jnp.int32(_CAP))
                return jnp.minimum(acc, p)

            pvec = lax.fori_loop(0, nvv, fp,
                                 jnp.full((_L,), _CAP, jnp.int32))
            p = jnp.min(pvec)
            pv = jnp.full((_L,), p, jnp.int32)
            kv = plsc.load_gather(ck, [pv])
            iv = plsc.load_gather(ci, [pv])
            irv = jnp.full((_L,), k < cnt)
            kv = jnp.where(irv, kv,
                           jnp.full((_L,), _KEY_NEG_INF, jnp.int32))
            iv = jnp.where(irv, iv, jnp.full((_L,), r + (k - cnt),
                                             jnp.int32))
            kk = jnp.full((_L,), k, jnp.int32)
            plsc.store_scatter(ok, [kk], kv, mask=lane0)
            plsc.store_scatter(oi, [kk], iv, mask=lane0)
            plsc.store_scatter(ck, [pv],
                               jnp.full((_L,), _SENTINEL, jnp.int32),
                               mask=lane0 & irv)
            return 0

        lax.fori_loop(0, K, sel, 0)
        for q in range(KP // _L):
            kq = ok[pl.ds(q * _L, _L)]
            flip = jnp.right_shift(kq, 31) & _KEY_XOR_MASK
            of[pl.ds(q * _L, _L)] = plsc.bitcast(kq ^ flip, jnp.float32)
        pltpu.sync_copy(of, outv_hbm.at[r])
        pltpu.sync_copy(oi, outi_hbm.at[r])
        return 0

    lax.fori_loop(0, N // _NW, row_body, 0)


def kernel(mentions, W, b):
    n, f = mentions.shape
    K = min(50, n)
    R = min(256, n)
    G = min(256, n)
    assert n % R == 0 and n % G == 0
    KP = ((K + _L - 1) // _L) * _L
    keys, thr = pl.pallas_call(
        functools.partial(_score_body, K, R, n, G),
        grid=(n // R,),
        in_specs=[
            pl.BlockSpec((R, f), lambda i: (i, 0)),
            pl.BlockSpec((n, f), lambda i: (0, 0)),
            pl.BlockSpec((f, f), lambda i: (0, 0)),
            pl.BlockSpec((1, f), lambda i: (0, 0)),
        ],
        out_specs=[
            pl.BlockSpec((R, n), lambda i: (i, 0)),
            pl.BlockSpec((R, _L), lambda i: (i, 0)),
        ],
        out_shape=[
            jax.ShapeDtypeStruct((n, n), jnp.int32),
            jax.ShapeDtypeStruct((n, _L), jnp.int32),
        ],
    )(mentions, mentions, W, b.reshape(1, f))

    sc = pl.kernel(
        functools.partial(_sc_body, K, n, KP),
        out_type=[
            jax.ShapeDtypeStruct((n, KP), jnp.float32),
            jax.ShapeDtypeStruct((n, KP), jnp.int32),
        ],
        mesh=plsc.VectorSubcoreMesh(core_axis_name="c",
                                    subcore_axis_name="s",
                                    num_cores=_NC, num_subcores=_NS),
        compiler_params=pltpu.CompilerParams(needs_layout_passes=False),
        scratch_types=[
            pltpu.VMEM((_L,), jnp.int32),
            pltpu.VMEM((n,), jnp.int32),
            pltpu.VMEM((_CAP,), jnp.int32),
            pltpu.VMEM((_CAP,), jnp.int32),
            pltpu.VMEM((KP,), jnp.int32),
            pltpu.VMEM((KP,), jnp.int32),
            pltpu.VMEM((KP,), jnp.float32),
        ],
    )
    outv, outi = sc(keys, thr)
    return outv[:, :K], outi[:, :K]


# speedup vs baseline: 9.7907x; 1.4429x over previous
import functools

import jax
import jax.numpy as jnp
from jax import lax
from jax.experimental import pallas as pl
from jax.experimental.pallas import tpu as pltpu
from jax.experimental.pallas import tpu_sc as plsc

NEG_INF = float("-inf")

_NC = 2
_NS = 16
_NW = _NC * _NS
_L = 16

_CAP = 128


def _score_body(K, R, N, G, m_rows_ref, m_all_ref, w_ref, b_ref,
                scores_ref, thr_ref):
    blk = pl.program_id(0)
    prec = lax.Precision.DEFAULT
    bilin = lax.dot_general(
        m_rows_ref[...], w_ref[...], (((1,), (1,)), ((), ())),
        preferred_element_type=jnp.float32, precision=prec) + b_ref[...]
    scores = lax.dot_general(
        bilin, m_all_ref[...], (((1,), (1,)), ((), ())),
        preferred_element_type=jnp.float32, precision=prec)
    col = lax.broadcasted_iota(jnp.int32, (R, N), 1)
    row = lax.broadcasted_iota(jnp.int32, (R, N), 0) + blk * R
    scores = jnp.where(col < row, scores, NEG_INF)
    scores_ref[...] = scores
    gm = scores[:, 0:G]
    for s in range(1, N // G):
        gm = jnp.maximum(gm, scores[:, s * G:(s + 1) * G])
    giota = lax.broadcasted_iota(jnp.int32, (R, G), 1)

    def tb(_, carry):
        gm, _ = carry
        m = jnp.max(gm, axis=1, keepdims=True)
        idx = jnp.min(jnp.where(gm == m, giota, jnp.int32(G)), axis=1,
                      keepdims=True)
        return jnp.where(giota == idx, NEG_INF, gm), m

    _, t = lax.fori_loop(0, K, tb, (gm, jnp.full((R, 1), NEG_INF,
                                                 jnp.float32)))
    thr_ref[...] = jnp.broadcast_to(t, (R, _L))


def _sc_body(N, scores_hbm, thr_hbm, candv_hbm, candc_hbm,
             rows_v, thrs_v, ov, oi, sem0, sem1):
    wid = lax.axis_index("s") * _NC + lax.axis_index("c")
    lanes = lax.iota(jnp.int32, _L)
    T = N // _NW
    sems = (sem0, sem1)

    def issue(t, slot):
        r = wid + t * _NW
        pltpu.async_copy(scores_hbm.at[r], rows_v.at[slot], sems[slot])
        pltpu.async_copy(thr_hbm.at[r], thrs_v.at[slot], sems[slot])

    def process(t, slot):
        r = wid + t * _NW
        pltpu.make_async_copy(scores_hbm.at[r], rows_v.at[slot],
                              sems[slot]).wait()
        pltpu.make_async_copy(thr_hbm.at[r], thrs_v.at[slot],
                              sems[slot]).wait()
        tvec = thrs_v[slot]
        rvec = jnp.full((_L,), r, jnp.int32)
        for q in range(_CAP // _L):
            ov[slot, pl.ds(q * _L, _L)] = jnp.full((_L,), NEG_INF,
                                                   jnp.float32)
            oi[slot, pl.ds(q * _L, _L)] = jnp.zeros((_L,), jnp.int32)
        nv = (r + _L - 1) // _L

        def filt(j, ptr):
            v = rows_v[slot, pl.ds(j * _L, _L)]
            colv = lanes + j * _L
            m = (v >= tvec) & (colv < rvec)
            s = plsc.cumsum(m.astype(jnp.int32))
            pos = jnp.minimum(ptr + s - 1, _CAP - 1)
            plsc.store_scatter(ov.at[slot], [pos], v, mask=m)
            plsc.store_scatter(oi.at[slot], [pos], colv, mask=m)
            return ptr + plsc.all_reduce_population_count(m)

        lax.fori_loop(0, nv, filt, jnp.zeros((_L,), jnp.int32))
        @pl.when(t + 2 < T)
        def _():
            issue(t + 2, slot)

        pltpu.sync_copy(ov.at[slot], candv_hbm.at[r])
        pltpu.sync_copy(oi.at[slot], candc_hbm.at[r])

    issue(0, 0)
    issue(1, 1)

    def pair(h, _):
        process(2 * h, 0)
        process(2 * h + 1, 1)
        return 0

    lax.fori_loop(0, T // 2, pair, 0)


def _sel_body(K, R, KP, candv_ref, candc_ref, out_v_ref, out_i_ref):
    blk = pl.program_id(0)
    candv = candv_ref[...]
    candc = candc_ref[...]
    siota = lax.broadcasted_iota(jnp.int32, (R, _CAP), 1)
    kio = lax.broadcasted_iota(jnp.int32, (R, KP), 1)
    cnt = jnp.sum((candv > NEG_INF).astype(jnp.int32), axis=1,
                  keepdims=True)

    def body(k, carry):
        candv, topv, topc = carry
        m = jnp.max(candv, axis=1, keepdims=True)
        eq = candv == m
        slot = jnp.min(jnp.where(eq, siota, jnp.int32(_CAP)), axis=1,
                       keepdims=True)
        hit = siota == slot
        colv = jnp.min(jnp.where(hit, candc, jnp.int32(0x7FFFFFFF)),
                       axis=1, keepdims=True)
        candv = jnp.where(hit, NEG_INF, candv)
        sel = kio == k
        topv = jnp.where(sel, m, topv)
        topc = jnp.where(sel, colv, topc)
        return candv, topv, topc

    _, topv, topc = lax.fori_loop(
        0, K, body,
        (candv, jnp.full((R, KP), NEG_INF, jnp.float32),
         jnp.zeros((R, KP), jnp.int32)))
    rvec = lax.broadcasted_iota(jnp.int32, (R, KP), 0) + blk * R
    tail = kio >= cnt
    outv = jnp.where(tail, NEG_INF, topv)
    outi = jnp.where(tail, rvec + kio - cnt, topc)
    out_v_ref[...] = outv[:, :K]
    out_i_ref[...] = outi[:, :K]


def kernel(mentions, W, b):
    n, f = mentions.shape
    K = min(50, n)
    R = min(256, n)
    G = min(256, n)
    assert n % R == 0 and n % G == 0 and n % _NW == 0
    KP = ((K + _L - 1) // _L) * _L
    scores, thr = pl.pallas_call(
        functools.partial(_score_body, K, R, n, G),
        grid=(n // R,),
        in_specs=[
            pl.BlockSpec((R, f), lambda i: (i, 0)),
            pl.BlockSpec((n, f), lambda i: (0, 0)),
            pl.BlockSpec((f, f), lambda i: (0, 0)),
            pl.BlockSpec((1, f), lambda i: (0, 0)),
        ],
        out_specs=[
            pl.BlockSpec((R, n), lambda i: (i, 0)),
            pl.BlockSpec((R, _L), lambda i: (i, 0)),
        ],
        out_shape=[
            jax.ShapeDtypeStruct((n, n), jnp.float32),
            jax.ShapeDtypeStruct((n, _L), jnp.float32),
        ],
    )(mentions, mentions, W, b.reshape(1, f))

    sc = pl.kernel(
        functools.partial(_sc_body, n),
        out_type=[
            jax.ShapeDtypeStruct((n, _CAP), jnp.float32),
            jax.ShapeDtypeStruct((n, _CAP), jnp.int32),
        ],
        mesh=plsc.VectorSubcoreMesh(core_axis_name="c",
                                    subcore_axis_name="s",
                                    num_cores=_NC, num_subcores=_NS),
        compiler_params=pltpu.CompilerParams(needs_layout_passes=False),
        scratch_types=[
            pltpu.VMEM((2, n), jnp.float32),
            pltpu.VMEM((2, _L), jnp.float32),
            pltpu.VMEM((2, _CAP), jnp.float32),
            pltpu.VMEM((2, _CAP), jnp.int32),
            pltpu.SemaphoreType.DMA,
            pltpu.SemaphoreType.DMA,
        ],
    )
    candv, candc = sc(scores, thr)

    R3 = min(512, n)
    out_v, out_i = pl.pallas_call(
        functools.partial(_sel_body, K, R3, KP),
        grid=(n // R3,),
        in_specs=[
            pl.BlockSpec((R3, _CAP), lambda i: (i, 0)),
            pl.BlockSpec((R3, _CAP), lambda i: (i, 0)),
        ],
        out_specs=[
            pl.BlockSpec((R3, K), lambda i: (i, 0)),
            pl.BlockSpec((R3, K), lambda i: (i, 0)),
        ],
        out_shape=[
            jax.ShapeDtypeStruct((n, K), jnp.float32),
            jax.ShapeDtypeStruct((n, K), jnp.int32),
        ],
    )(candv, candc)
    return out_v, out_i


# speedup vs baseline: 10.2599x; 1.0479x over previous
import functools

import jax
import jax.numpy as jnp
from jax import lax
from jax.experimental import pallas as pl
from jax.experimental.pallas import tpu as pltpu
from jax.experimental.pallas import tpu_sc as plsc

NEG_INF = float("-inf")

_NC = 2
_NS = 16
_NW = _NC * _NS
_L = 16

_CAP = 128


def _score_body(K, R, N, G, m_rows_ref, m_all_ref, w_ref, b_ref,
                scores_ref, thr_ref):
    blk = pl.program_id(0)
    prec = lax.Precision.DEFAULT
    bilin = lax.dot_general(
        m_rows_ref[...], w_ref[...], (((1,), (1,)), ((), ())),
        preferred_element_type=jnp.float32, precision=prec) + b_ref[...]
    scores = lax.dot_general(
        bilin, m_all_ref[...], (((1,), (1,)), ((), ())),
        preferred_element_type=jnp.float32, precision=prec)
    col = lax.broadcasted_iota(jnp.int32, (R, N), 1)
    row = lax.broadcasted_iota(jnp.int32, (R, N), 0) + blk * R
    scores = jnp.where(col < row, scores, NEG_INF)
    scores_ref[...] = scores
    gm = scores[:, 0:G]
    for s in range(1, N // G):
        gm = jnp.maximum(gm, scores[:, s * G:(s + 1) * G])
    giota = lax.broadcasted_iota(jnp.int32, (R, G), 1)

    def tb(_, carry):
        gm, _ = carry
        m = jnp.max(gm, axis=1, keepdims=True)
        idx = jnp.min(jnp.where(gm == m, giota, jnp.int32(G)), axis=1,
                      keepdims=True)
        return jnp.where(giota == idx, NEG_INF, gm), m

    _, t = lax.fori_loop(0, K, tb, (gm, jnp.full((R, 1), NEG_INF,
                                                 jnp.float32)))
    thr_ref[...] = jnp.broadcast_to(t, (R, _L))


def _sc_body(N, scores_hbm, thr_hbm, candv_hbm, candc_hbm,
             rows_v, thrs_v, ov, oi, sem0, sem1, osem0, osem1):
    wid = lax.axis_index("s") * _NC + lax.axis_index("c")
    lanes = lax.iota(jnp.int32, _L)
    T = N // _NW
    sems = (sem0, sem1)
    osems = (osem0, osem1)

    def issue(t, slot):
        r = wid + t * _NW
        pltpu.async_copy(scores_hbm.at[r], rows_v.at[slot], sems[slot])
        pltpu.async_copy(thr_hbm.at[r], thrs_v.at[slot], sems[slot])

    def drain_out(slot, r):
        pltpu.make_async_copy(ov.at[slot], candv_hbm.at[r],
                              osems[slot]).wait()
        pltpu.make_async_copy(oi.at[slot], candc_hbm.at[r],
                              osems[slot]).wait()

    def process(t, slot):
        r = wid + t * _NW
        pltpu.make_async_copy(scores_hbm.at[r], rows_v.at[slot],
                              sems[slot]).wait()
        pltpu.make_async_copy(thr_hbm.at[r], thrs_v.at[slot],
                              sems[slot]).wait()
        @pl.when(t >= 2)
        def _():
            drain_out(slot, r)

        tvec = thrs_v[slot]
        rvec = jnp.full((_L,), r, jnp.int32)
        for q in range(_CAP // _L):
            ov[slot, pl.ds(q * _L, _L)] = jnp.full((_L,), NEG_INF,
                                                   jnp.float32)
            oi[slot, pl.ds(q * _L, _L)] = jnp.zeros((_L,), jnp.int32)
        nfull = r // _L
        ng = nfull // 4

        def grp(g, ptr):
            for u in range(4):
                jj = g * 4 + u
                v = rows_v[slot, pl.ds(jj * _L, _L)]
                m = v >= tvec
                s = plsc.cumsum(m.astype(jnp.int32))
                pos = jnp.minimum(ptr + s - 1, _CAP - 1)
                plsc.store_scatter(ov.at[slot], [pos], v, mask=m)
                plsc.store_scatter(oi.at[slot], [pos], lanes + jj * _L,
                                   mask=m)
                ptr = ptr + plsc.all_reduce_population_count(m)
            return ptr

        ptr = lax.fori_loop(0, ng, grp, jnp.zeros((_L,), jnp.int32))
        nv = (r + _L - 1) // _L

        def rem(j, ptr):
            v = rows_v[slot, pl.ds(j * _L, _L)]
            colv = lanes + j * _L
            m = (v >= tvec) & (colv < rvec)
            s = plsc.cumsum(m.astype(jnp.int32))
            pos = jnp.minimum(ptr + s - 1, _CAP - 1)
            plsc.store_scatter(ov.at[slot], [pos], v, mask=m)
            plsc.store_scatter(oi.at[slot], [pos], colv, mask=m)
            return ptr + plsc.all_reduce_population_count(m)

        lax.fori_loop(ng * 4, nv, rem, ptr)
        @pl.when(t + 2 < T)
        def _():
            issue(t + 2, slot)

        pltpu.async_copy(ov.at[slot], candv_hbm.at[r], osems[slot])
        pltpu.async_copy(oi.at[slot], candc_hbm.at[r], osems[slot])

    issue(0, 0)
    issue(1, 1)

    def pair(h, _):
        process(2 * h, 0)
        process(2 * h + 1, 1)
        return 0

    lax.fori_loop(0, T // 2, pair, 0)
    drain_out(0, wid + (T - 2) * _NW)
    drain_out(1, wid + (T - 1) * _NW)


def _sel_body(K, R, KP, candv_ref, candc_ref, out_v_ref, out_i_ref):
    blk = pl.program_id(0)
    candv = candv_ref[...]
    candc = candc_ref[...]
    siota = lax.broadcasted_iota(jnp.int32, (R, _CAP), 1)
    kio = lax.broadcasted_iota(jnp.int32, (R, KP), 1)
    cnt = jnp.sum((candv > NEG_INF).astype(jnp.int32), axis=1,
                  keepdims=True)

    def body(k, carry):
        candv, topv, topc = carry
        m = jnp.max(candv, axis=1, keepdims=True)
        eq = candv == m
        slot = jnp.min(jnp.where(eq, siota, jnp.int32(_CAP)), axis=1,
                       keepdims=True)
        hit = siota == slot
        colv = jnp.min(jnp.where(hit, candc, jnp.int32(0x7FFFFFFF)),
                       axis=1, keepdims=True)
        candv = jnp.where(hit, NEG_INF, candv)
        sel = kio == k
        topv = jnp.where(sel, m, topv)
        topc = jnp.where(sel, colv, topc)
        return candv, topv, topc

    _, topv, topc = lax.fori_loop(
        0, K, body,
        (candv, jnp.full((R, KP), NEG_INF, jnp.float32),
         jnp.zeros((R, KP), jnp.int32)))
    rvec = lax.broadcasted_iota(jnp.int32, (R, KP), 0) + blk * R
    tail = kio >= cnt
    outv = jnp.where(tail, NEG_INF, topv)
    outi = jnp.where(tail, rvec + kio - cnt, topc)
    out_v_ref[...] = outv[:, :K]
    out_i_ref[...] = outi[:, :K]


def kernel(mentions, W, b):
    n, f = mentions.shape
    K = min(50, n)
    R = min(256, n)
    G = min(256, n)
    assert n % R == 0 and n % G == 0 and n % _NW == 0
    KP = ((K + _L - 1) // _L) * _L
    scores, thr = pl.pallas_call(
        functools.partial(_score_body, K, R, n, G),
        grid=(n // R,),
        in_specs=[
            pl.BlockSpec((R, f), lambda i: (i, 0)),
            pl.BlockSpec((n, f), lambda i: (0, 0)),
            pl.BlockSpec((f, f), lambda i: (0, 0)),
            pl.BlockSpec((1, f), lambda i: (0, 0)),
        ],
        out_specs=[
            pl.BlockSpec((R, n), lambda i: (i, 0)),
            pl.BlockSpec((R, _L), lambda i: (i, 0)),
        ],
        out_shape=[
            jax.ShapeDtypeStruct((n, n), jnp.float32),
            jax.ShapeDtypeStruct((n, _L), jnp.float32),
        ],
    )(mentions, mentions, W, b.reshape(1, f))

    sc = pl.kernel(
        functools.partial(_sc_body, n),
        out_type=[
            jax.ShapeDtypeStruct((n, _CAP), jnp.float32),
            jax.ShapeDtypeStruct((n, _CAP), jnp.int32),
        ],
        mesh=plsc.VectorSubcoreMesh(core_axis_name="c",
                                    subcore_axis_name="s",
                                    num_cores=_NC, num_subcores=_NS),
        compiler_params=pltpu.CompilerParams(needs_layout_passes=False),
        scratch_types=[
            pltpu.VMEM((2, n), jnp.float32),
            pltpu.VMEM((2, _L), jnp.float32),
            pltpu.VMEM((2, _CAP), jnp.float32),
            pltpu.VMEM((2, _CAP), jnp.int32),
            pltpu.SemaphoreType.DMA,
            pltpu.SemaphoreType.DMA,
            pltpu.SemaphoreType.DMA,
            pltpu.SemaphoreType.DMA,
        ],
    )
    candv, candc = sc(scores, thr)

    R3 = min(512, n)
    out_v, out_i = pl.pallas_call(
        functools.partial(_sel_body, K, R3, KP),
        grid=(n // R3,),
        in_specs=[
            pl.BlockSpec((R3, _CAP), lambda i: (i, 0)),
            pl.BlockSpec((R3, _CAP), lambda i: (i, 0)),
        ],
        out_specs=[
            pl.BlockSpec((R3, K), lambda i: (i, 0)),
            pl.BlockSpec((R3, K), lambda i: (i, 0)),
        ],
        out_shape=[
            jax.ShapeDtypeStruct((n, K), jnp.float32),
            jax.ShapeDtypeStruct((n, K), jnp.int32),
        ],
    )(candv, candc)
    return out_v, out_i


# speedup vs baseline: 11.9633x; 1.1660x over previous
import functools

import jax
import jax.numpy as jnp
from jax import lax
from jax.experimental import pallas as pl
from jax.experimental.pallas import tpu as pltpu
from jax.experimental.pallas import tpu_sc as plsc

NEG_INF = float("-inf")

_NC = 2
_NS = 16
_NW = _NC * _NS
_L = 16

_CAP = 128


def _score_body(K, R, N, G, m_rows_ref, m_all_ref, w_ref, b_ref,
                scores_ref, thr_ref):
    blk = pl.program_id(0)
    prec = lax.Precision.DEFAULT
    bilin = lax.dot_general(
        m_rows_ref[...], w_ref[...], (((1,), (1,)), ((), ())),
        preferred_element_type=jnp.float32, precision=prec) + b_ref[...]
    scores = lax.dot_general(
        bilin, m_all_ref[...], (((1,), (1,)), ((), ())),
        preferred_element_type=jnp.float32, precision=prec)
    col = lax.broadcasted_iota(jnp.int32, (R, N), 1)
    row = lax.broadcasted_iota(jnp.int32, (R, N), 0) + blk * R
    scores = jnp.where(col < row, scores, NEG_INF)
    scores_ref[...] = scores
    gm = scores[:, 0:G]
    for s in range(1, N // G):
        gm = jnp.maximum(gm, scores[:, s * G:(s + 1) * G])
    giota = lax.broadcasted_iota(jnp.int32, (R, G), 1)

    def tb(_, carry):
        gm, _ = carry
        m = jnp.max(gm, axis=1, keepdims=True)
        idx = jnp.min(jnp.where(gm == m, giota, jnp.int32(G)), axis=1,
                      keepdims=True)
        return jnp.where(giota == idx, NEG_INF, gm), m

    _, t = lax.fori_loop(0, K, tb, (gm, jnp.full((R, 1), NEG_INF,
                                                 jnp.float32)))
    thr_ref[...] = jnp.broadcast_to(t, (R, _L))


def _sc_body(N, scores_hbm, thr_hbm, candv_hbm, candc_hbm,
             rows_v, thrs_v, ov, oi, sem0, sem1, osem0, osem1):
    wid = lax.axis_index("s") * _NC + lax.axis_index("c")
    lanes = lax.iota(jnp.int32, _L)
    T = N // _NW
    sems = (sem0, sem1)
    osems = (osem0, osem1)

    def issue(t, slot):
        r = wid + t * _NW
        pltpu.async_copy(scores_hbm.at[r], rows_v.at[slot], sems[slot])
        pltpu.async_copy(thr_hbm.at[r], thrs_v.at[slot], sems[slot])

    def drain_out(slot, r):
        pltpu.make_async_copy(ov.at[slot], candv_hbm.at[r],
                              osems[slot]).wait()
        pltpu.make_async_copy(oi.at[slot], candc_hbm.at[r],
                              osems[slot]).wait()

    def process(t, slot):
        r = wid + t * _NW
        pltpu.make_async_copy(scores_hbm.at[r], rows_v.at[slot],
                              sems[slot]).wait()
        pltpu.make_async_copy(thr_hbm.at[r], thrs_v.at[slot],
                              sems[slot]).wait()
        @pl.when(t >= 2)
        def _():
            drain_out(slot, r)

        tvec = thrs_v[slot]
        rvec = jnp.full((_L,), r, jnp.int32)
        for q in range(_CAP // _L):
            ov[slot, pl.ds(q * _L, _L)] = jnp.full((_L,), NEG_INF,
                                                   jnp.float32)
            oi[slot, pl.ds(q * _L, _L)] = jnp.zeros((_L,), jnp.int32)
        nfull = r // _L
        ng = nfull // 4

        def grp(g, ptr):
            j0 = g * 4
            vs, ms, cs = [], [], []
            tot = jnp.zeros((_L,), jnp.int32)
            for u in range(4):
                v = rows_v[slot, pl.ds((j0 + u) * _L, _L)]
                m = v >= tvec
                c = plsc.all_reduce_population_count(m)
                vs.append(v)
                ms.append(m)
                cs.append(c)
                tot = tot + c

            def slow(p):
                for u in range(4):
                    def app(q, u=u):
                        s = plsc.cumsum(ms[u].astype(jnp.int32))
                        pos = jnp.minimum(q + s - 1, _CAP - 1)
                        plsc.store_scatter(ov.at[slot], [pos], vs[u],
                                           mask=ms[u])
                        plsc.store_scatter(oi.at[slot], [pos],
                                           lanes + (j0 + u) * _L,
                                           mask=ms[u])
                        return q + cs[u]

                    p = lax.cond(cs[u][0] > 0, app, lambda q: q, p)
                return p

            return lax.cond(tot[0] > 0, slow, lambda q: q, ptr)

        ptr = lax.fori_loop(0, ng, grp, jnp.zeros((_L,), jnp.int32))
        nv = (r + _L - 1) // _L

        def rem(j, ptr):
            v = rows_v[slot, pl.ds(j * _L, _L)]
            colv = lanes + j * _L
            m = (v >= tvec) & (colv < rvec)
            s = plsc.cumsum(m.astype(jnp.int32))
            pos = jnp.minimum(ptr + s - 1, _CAP - 1)
            plsc.store_scatter(ov.at[slot], [pos], v, mask=m)
            plsc.store_scatter(oi.at[slot], [pos], colv, mask=m)
            return ptr + plsc.all_reduce_population_count(m)

        lax.fori_loop(ng * 4, nv, rem, ptr)
        @pl.when(t + 2 < T)
        def _():
            issue(t + 2, slot)

        pltpu.async_copy(ov.at[slot], candv_hbm.at[r], osems[slot])
        pltpu.async_copy(oi.at[slot], candc_hbm.at[r], osems[slot])

    issue(0, 0)
    issue(1, 1)

    def pair(h, _):
        process(2 * h, 0)
        process(2 * h + 1, 1)
        return 0

    lax.fori_loop(0, T // 2, pair, 0)
    drain_out(0, wid + (T - 2) * _NW)
    drain_out(1, wid + (T - 1) * _NW)


def _sel_body(K, R, KP, candv_ref, candc_ref, out_v_ref, out_i_ref):
    blk = pl.program_id(0)
    candv = candv_ref[...]
    candc = candc_ref[...]
    siota = lax.broadcasted_iota(jnp.int32, (R, _CAP), 1)
    kio = lax.broadcasted_iota(jnp.int32, (R, KP), 1)
    cnt = jnp.sum((candv > NEG_INF).astype(jnp.int32), axis=1,
                  keepdims=True)

    def body(k, carry):
        candv, topv, topc = carry
        m = jnp.max(candv, axis=1, keepdims=True)
        eq = candv == m
        slot = jnp.min(jnp.where(eq, siota, jnp.int32(_CAP)), axis=1,
                       keepdims=True)
        hit = siota == slot
        colv = jnp.min(jnp.where(hit, candc, jnp.int32(0x7FFFFFFF)),
                       axis=1, keepdims=True)
        candv = jnp.where(hit, NEG_INF, candv)
        sel = kio == k
        topv = jnp.where(sel, m, topv)
        topc = jnp.where(sel, colv, topc)
        return candv, topv, topc

    _, topv, topc = lax.fori_loop(
        0, K, body,
        (candv, jnp.full((R, KP), NEG_INF, jnp.float32),
         jnp.zeros((R, KP), jnp.int32)))
    rvec = lax.broadcasted_iota(jnp.int32, (R, KP), 0) + blk * R
    tail = kio >= cnt
    outv = jnp.where(tail, NEG_INF, topv)
    outi = jnp.where(tail, rvec + kio - cnt, topc)
    out_v_ref[...] = outv[:, :K]
    out_i_ref[...] = outi[:, :K]


def kernel(mentions, W, b):
    n, f = mentions.shape
    K = min(50, n)
    R = min(256, n)
    G = min(256, n)
    assert n % R == 0 and n % G == 0 and n % _NW == 0
    KP = ((K + _L - 1) // _L) * _L
    scores, thr = pl.pallas_call(
        functools.partial(_score_body, K, R, n, G),
        grid=(n // R,),
        in_specs=[
            pl.BlockSpec((R, f), lambda i: (i, 0)),
            pl.BlockSpec((n, f), lambda i: (0, 0)),
            pl.BlockSpec((f, f), lambda i: (0, 0)),
            pl.BlockSpec((1, f), lambda i: (0, 0)),
        ],
        out_specs=[
            pl.BlockSpec((R, n), lambda i: (i, 0)),
            pl.BlockSpec((R, _L), lambda i: (i, 0)),
        ],
        out_shape=[
            jax.ShapeDtypeStruct((n, n), jnp.float32),
            jax.ShapeDtypeStruct((n, _L), jnp.float32),
        ],
    )(mentions, mentions, W, b.reshape(1, f))

    sc = pl.kernel(
        functools.partial(_sc_body, n),
        out_type=[
            jax.ShapeDtypeStruct((n, _CAP), jnp.float32),
            jax.ShapeDtypeStruct((n, _CAP), jnp.int32),
        ],
        mesh=plsc.VectorSubcoreMesh(core_axis_name="c",
                                    subcore_axis_name="s",
                                    num_cores=_NC, num_subcores=_NS),
        compiler_params=pltpu.CompilerParams(needs_layout_passes=False),
        scratch_types=[
            pltpu.VMEM((2, n), jnp.float32),
            pltpu.VMEM((2, _L), jnp.float32),
            pltpu.VMEM((2, _CAP), jnp.float32),
            pltpu.VMEM((2, _CAP), jnp.int32),
            pltpu.SemaphoreType.DMA,
            pltpu.SemaphoreType.DMA,
            pltpu.SemaphoreType.DMA,
            pltpu.SemaphoreType.DMA,
        ],
    )
    candv, candc = sc(scores, thr)

    R3 = min(512, n)
    out_v, out_i = pl.pallas_call(
        functools.partial(_sel_body, K, R3, KP),
        grid=(n // R3,),
        in_specs=[
            pl.BlockSpec((R3, _CAP), lambda i: (i, 0)),
            pl.BlockSpec((R3, _CAP), lambda i: (i, 0)),
        ],
        out_specs=[
            pl.BlockSpec((R3, K), lambda i: (i, 0)),
            pl.BlockSpec((R3, K), lambda i: (i, 0)),
        ],
        out_shape=[
            jax.ShapeDtypeStruct((n, K), jnp.float32),
            jax.ShapeDtypeStruct((n, K), jnp.int32),
        ],
    )(candv, candc)
    return out_v, out_i


# speedup vs baseline: 14.3796x; 1.2020x over previous
import functools

import jax
import jax.numpy as jnp
from jax import lax
from jax.experimental import pallas as pl
from jax.experimental.pallas import tpu as pltpu
from jax.experimental.pallas import tpu_sc as plsc

NEG_INF = float("-inf")

_NC = 2
_NS = 16
_NW = _NC * _NS
_L = 16

_CAP = 128


def _score_body(K, R, N, G, base, m_rows_ref, m_all_ref, w_ref, b_ref,
                scores_ref, thr_ref):
    blk = pl.program_id(0)
    prec = lax.Precision.DEFAULT
    bilin = lax.dot_general(
        m_rows_ref[...], w_ref[...], (((1,), (1,)), ((), ())),
        preferred_element_type=jnp.float32, precision=prec) + b_ref[...]
    scores = lax.dot_general(
        bilin, m_all_ref[...], (((1,), (1,)), ((), ())),
        preferred_element_type=jnp.float32, precision=prec)
    col = lax.broadcasted_iota(jnp.int32, (R, N), 1)
    row = lax.broadcasted_iota(jnp.int32, (R, N), 0) + (base + blk * R)
    scores = jnp.where(col < row, scores, NEG_INF)
    scores_ref[...] = scores
    gm = scores[:, 0:G]
    for s in range(1, N // G):
        gm = jnp.maximum(gm, scores[:, s * G:(s + 1) * G])
    giota = lax.broadcasted_iota(jnp.int32, (R, G), 1)

    def tb(_, carry):
        gm, _ = carry
        m = jnp.max(gm, axis=1, keepdims=True)
        idx = jnp.min(jnp.where(gm == m, giota, jnp.int32(G)), axis=1,
                      keepdims=True)
        return jnp.where(giota == idx, NEG_INF, gm), m

    _, t = lax.fori_loop(0, K, tb, (gm, jnp.full((R, 1), NEG_INF,
                                                 jnp.float32)))
    thr_ref[...] = jnp.broadcast_to(t, (R, _L))


def _sc_body(CR, base, scores_hbm, thr_hbm, candv_hbm, candc_hbm,
             rows_v, thrs_v, ov, oi, sem0, sem1, osem0, osem1):
    wid = lax.axis_index("s") * _NC + lax.axis_index("c")
    lanes = lax.iota(jnp.int32, _L)
    T = CR // _NW
    sems = (sem0, sem1)
    osems = (osem0, osem1)

    def issue(t, slot):
        lr = wid + t * _NW
        pltpu.async_copy(scores_hbm.at[lr], rows_v.at[slot], sems[slot])
        pltpu.async_copy(thr_hbm.at[lr], thrs_v.at[slot], sems[slot])

    def drain_out(slot, lr):
        pltpu.make_async_copy(ov.at[slot], candv_hbm.at[lr],
                              osems[slot]).wait()
        pltpu.make_async_copy(oi.at[slot], candc_hbm.at[lr],
                              osems[slot]).wait()

    def process(t, slot):
        lr = wid + t * _NW
        r = base + lr
        pltpu.make_async_copy(scores_hbm.at[lr], rows_v.at[slot],
                              sems[slot]).wait()
        pltpu.make_async_copy(thr_hbm.at[lr], thrs_v.at[slot],
                              sems[slot]).wait()
        @pl.when(t >= 2)
        def _():
            drain_out(slot, lr)

        tvec = thrs_v[slot]
        rvec = jnp.full((_L,), r, jnp.int32)
        for q in range(_CAP // _L):
            ov[slot, pl.ds(q * _L, _L)] = jnp.full((_L,), NEG_INF,
                                                   jnp.float32)
            oi[slot, pl.ds(q * _L, _L)] = jnp.zeros((_L,), jnp.int32)
        nfull = r // _L
        ng = nfull // 4

        def grp(g, ptr):
            j0 = g * 4
            vs, ms, cs = [], [], []
            tot = jnp.zeros((_L,), jnp.int32)
            for u in range(4):
                v = rows_v[slot, pl.ds((j0 + u) * _L, _L)]
                m = v >= tvec
                c = plsc.all_reduce_population_count(m)
                vs.append(v)
                ms.append(m)
                cs.append(c)
                tot = tot + c

            def slow(p):
                for u in range(4):
                    def app(q, u=u):
                        s = plsc.cumsum(ms[u].astype(jnp.int32))
                        pos = jnp.minimum(q + s - 1, _CAP - 1)
                        plsc.store_scatter(ov.at[slot], [pos], vs[u],
                                           mask=ms[u])
                        plsc.store_scatter(oi.at[slot], [pos],
                                           lanes + (j0 + u) * _L,
                                           mask=ms[u])
                        return q + cs[u]

                    p = lax.cond(cs[u][0] > 0, app, lambda q: q, p)
                return p

            return lax.cond(tot[0] > 0, slow, lambda q: q, ptr)

        ptr = lax.fori_loop(0, ng, grp, jnp.zeros((_L,), jnp.int32))
        nv = (r + _L - 1) // _L

        def rem(j, ptr):
            v = rows_v[slot, pl.ds(j * _L, _L)]
            colv = lanes + j * _L
            m = (v >= tvec) & (colv < rvec)
            s = plsc.cumsum(m.astype(jnp.int32))
            pos = jnp.minimum(ptr + s - 1, _CAP - 1)
            plsc.store_scatter(ov.at[slot], [pos], v, mask=m)
            plsc.store_scatter(oi.at[slot], [pos], colv, mask=m)
            return ptr + plsc.all_reduce_population_count(m)

        lax.fori_loop(ng * 4, nv, rem, ptr)
        @pl.when(t + 2 < T)
        def _():
            issue(t + 2, slot)

        pltpu.async_copy(ov.at[slot], candv_hbm.at[lr], osems[slot])
        pltpu.async_copy(oi.at[slot], candc_hbm.at[lr], osems[slot])

    issue(0, 0)
    issue(1, 1)

    def pair(h, _):
        process(2 * h, 0)
        process(2 * h + 1, 1)
        return 0

    lax.fori_loop(0, T // 2, pair, 0)
    drain_out(0, wid + (T - 2) * _NW)
    drain_out(1, wid + (T - 1) * _NW)


def _sel_body(K, R, KP, candv_ref, candc_ref, out_v_ref, out_i_ref):
    blk = pl.program_id(0)
    candv = candv_ref[...]
    candc = candc_ref[...]
    siota = lax.broadcasted_iota(jnp.int32, (R, _CAP), 1)
    kio = lax.broadcasted_iota(jnp.int32, (R, KP), 1)
    cnt = jnp.sum((candv > NEG_INF).astype(jnp.int32), axis=1,
                  keepdims=True)

    def body(k, carry):
        candv, topv, topc = carry
        m = jnp.max(candv, axis=1, keepdims=True)
        eq = candv == m
        slot = jnp.min(jnp.where(eq, siota, jnp.int32(_CAP)), axis=1,
                       keepdims=True)
        hit = siota == slot
        colv = jnp.min(jnp.where(hit, candc, jnp.int32(0x7FFFFFFF)),
                       axis=1, keepdims=True)
        candv = jnp.where(hit, NEG_INF, candv)
        sel = kio == k
        topv = jnp.where(sel, m, topv)
        topc = jnp.where(sel, colv, topc)
        return candv, topv, topc

    _, topv, topc = lax.fori_loop(
        0, K, body,
        (candv, jnp.full((R, KP), NEG_INF, jnp.float32),
         jnp.zeros((R, KP), jnp.int32)))
    rvec = lax.broadcasted_iota(jnp.int32, (R, KP), 0) + blk * R
    tail = kio >= cnt
    outv = jnp.where(tail, NEG_INF, topv)
    outi = jnp.where(tail, rvec + kio - cnt, topc)
    out_v_ref[...] = outv[:, :K]
    out_i_ref[...] = outi[:, :K]


def kernel(mentions, W, b):
    n, f = mentions.shape
    K = min(50, n)
    R = min(256, n)
    G = min(256, n)
    assert n % R == 0 and n % G == 0 and n % _NW == 0
    KP = ((K + _L - 1) // _L) * _L
    NCH = 4 if n % (4 * 2 * _NW) == 0 else 1
    CR = n // NCH
    b2 = b.reshape(1, f)

    def stage1(base):
        return pl.pallas_call(
            functools.partial(_score_body, K, R, n, G, base),
            grid=(CR // R,),
            in_specs=[
                pl.BlockSpec((R, f), lambda i: (i, 0)),
                pl.BlockSpec((n, f), lambda i: (0, 0)),
                pl.BlockSpec((f, f), lambda i: (0, 0)),
                pl.BlockSpec((1, f), lambda i: (0, 0)),
            ],
            out_specs=[
                pl.BlockSpec((R, n), lambda i: (i, 0)),
                pl.BlockSpec((R, _L), lambda i: (i, 0)),
            ],
            out_shape=[
                jax.ShapeDtypeStruct((CR, n), jnp.float32),
                jax.ShapeDtypeStruct((CR, _L), jnp.float32),
            ],
        )(lax.dynamic_slice_in_dim(mentions, base, CR, 0), mentions, W, b2)

    def sc_chunk(base):
        return pl.kernel(
            functools.partial(_sc_body, CR, base),
            out_type=[
                jax.ShapeDtypeStruct((CR, _CAP), jnp.float32),
                jax.ShapeDtypeStruct((CR, _CAP), jnp.int32),
            ],
            mesh=plsc.VectorSubcoreMesh(core_axis_name="c",
                                        subcore_axis_name="s",
                                        num_cores=_NC, num_subcores=_NS),
            compiler_params=pltpu.CompilerParams(
                needs_layout_passes=False),
            scratch_types=[
                pltpu.VMEM((2, n), jnp.float32),
                pltpu.VMEM((2, _L), jnp.float32),
                pltpu.VMEM((2, _CAP), jnp.float32),
                pltpu.VMEM((2, _CAP), jnp.int32),
                pltpu.SemaphoreType.DMA,
                pltpu.SemaphoreType.DMA,
                pltpu.SemaphoreType.DMA,
                pltpu.SemaphoreType.DMA,
            ],
        )

    parts = {}
    for c in reversed(range(NCH)):
        base = c * CR
        scores_c, thr_c = stage1(base)
        parts[c] = sc_chunk(base)(scores_c, thr_c)
    candv = jnp.concatenate([parts[c][0] for c in range(NCH)], axis=0)
    candc = jnp.concatenate([parts[c][1] for c in range(NCH)], axis=0)

    R3 = min(512, n)
    out_v, out_i = pl.pallas_call(
        functools.partial(_sel_body, K, R3, KP),
        grid=(n // R3,),
        in_specs=[
            pl.BlockSpec((R3, _CAP), lambda i: (i, 0)),
            pl.BlockSpec((R3, _CAP), lambda i: (i, 0)),
        ],
        out_specs=[
            pl.BlockSpec((R3, K), lambda i: (i, 0)),
            pl.BlockSpec((R3, K), lambda i: (i, 0)),
        ],
        out_shape=[
            jax.ShapeDtypeStruct((n, K), jnp.float32),
            jax.ShapeDtypeStruct((n, K), jnp.int32),
        ],
    )(candv, candc)
    return out_v, out_i
